# baseline (speedup 1.0000x reference)
import dataclasses
import functools

import jax
import jax.numpy as jnp
from jax import lax
from jax.experimental import pallas as pl
from jax.experimental.pallas import tpu as pltpu
from jax.experimental.pallas import tpu_sc as plsc

N = 50000
E = 800000
H = 32
NC = 10
B = 64

NUM_CORES = 2
NUM_SUBCORES = 16
NW = NUM_CORES * NUM_SUBCORES
E_PER_W = E // NW
GATHER_CHUNK = 1000
POS_CHUNK = 5000
SCAT_CHUNK = 4000
EB = 6400

_SC_PARAMS = pltpu.CompilerParams(use_tc_tiling_on_sc=False)
if "needs_layout_passes" in pltpu.CompilerParams.__dataclass_fields__:
  _SC_PARAMS = dataclasses.replace(_SC_PARAMS, needs_layout_passes=False)


def _vmesh():
  return plsc.VectorSubcoreMesh(core_axis_name="c", subcore_axis_name="s")


def _wid():
  return lax.axis_index("s") * NUM_CORES + lax.axis_index("c")


def _sc_gather_rows(table, idx):
  D = table.shape[1]

  @functools.partial(
      pl.kernel,
      out_type=jax.ShapeDtypeStruct((E, D), jnp.float32),
      mesh=_vmesh(),
      compiler_params=_SC_PARAMS,
      scratch_types=[
          pltpu.VMEM((GATHER_CHUNK,), jnp.int32),
          pltpu.VMEM((GATHER_CHUNK, D), jnp.float32),
          pltpu.SemaphoreType.DMA,
      ],
  )
  def gather_kernel(table_hbm, idx_hbm, out_hbm, idx_v, rows_v, sem):
    base = _wid() * E_PER_W

    @pl.loop(0, E_PER_W // GATHER_CHUNK)
    def _(j):
      off = base + j * GATHER_CHUNK
      pltpu.sync_copy(idx_hbm.at[pl.ds(off, GATHER_CHUNK)], idx_v)
      pltpu.async_copy(table_hbm.at[idx_v], rows_v, sem).wait()
      pltpu.sync_copy(rows_v, out_hbm.at[pl.ds(off, GATHER_CHUNK)])

  return gather_kernel(table, idx)


def _sc_gather_pos(p8, src, dst):
  @functools.partial(
      pl.kernel,
      out_type=(
          jax.ShapeDtypeStruct((E, 8), jnp.float32),
          jax.ShapeDtypeStruct((E, 8), jnp.float32),
      ),
      mesh=_vmesh(),
      compiler_params=_SC_PARAMS,
      scratch_types=[
          pltpu.VMEM((POS_CHUNK,), jnp.int32),
          pltpu.VMEM((POS_CHUNK,), jnp.int32),
          pltpu.VMEM((POS_CHUNK, 8), jnp.float32),
          pltpu.VMEM((POS_CHUNK, 8), jnp.float32),
          pltpu.SemaphoreType.DMA,
          pltpu.SemaphoreType.DMA,
      ],
  )
  def gather_kernel(p8_hbm, src_hbm, dst_hbm, ps_hbm, pd_hbm, si_v, di_v,
                    ps_v, pd_v, sem1, sem2):
    base = _wid() * E_PER_W

    @pl.loop(0, E_PER_W // POS_CHUNK)
    def _(j):
      off = base + j * POS_CHUNK
      pltpu.sync_copy(src_hbm.at[pl.ds(off, POS_CHUNK)], si_v)
      pltpu.sync_copy(dst_hbm.at[pl.ds(off, POS_CHUNK)], di_v)
      pltpu.async_copy(p8_hbm.at[si_v], ps_v, sem1).wait()
      pltpu.async_copy(p8_hbm.at[di_v], pd_v, sem2).wait()
      pltpu.sync_copy(ps_v, ps_hbm.at[pl.ds(off, POS_CHUNK)])
      pltpu.sync_copy(pd_v, pd_hbm.at[pl.ds(off, POS_CHUNK)])

  return gather_kernel(p8, src, dst)


def _scatter_group(acc_v, tmp_v, idx, vals, ids):
  plsc.store_scatter(tmp_v, [idx], ids)
  got = plsc.load_gather(tmp_v, [idx])
  own = got == ids
  cur = plsc.load_gather(acc_v, [idx])
  plsc.store_scatter(acc_v, [idx], jnp.maximum(cur, vals), mask=own)
  pend = jnp.where(own, 0, 1)

  def cond(p):
    return lax.reduce_max(p, (0,)) > 0

  def body(p):
    pm = p > 0
    plsc.store_scatter(tmp_v, [idx], ids, mask=pm)
    g2 = plsc.load_gather(tmp_v, [idx])
    own2 = pm & (g2 == ids)
    c2 = plsc.load_gather(acc_v, [idx])
    plsc.store_scatter(acc_v, [idx], jnp.maximum(c2, vals), mask=own2)
    return jnp.where(own2, 0, p)

  lax.while_loop(cond, body, pend)


def _sc_scatter_max(mT, dst):
  @functools.partial(
      pl.kernel,
      out_type=jax.ShapeDtypeStruct((H, N), jnp.float32),
      mesh=_vmesh(),
      compiler_params=_SC_PARAMS,
      scratch_types=[
          pltpu.VMEM((N,), jnp.float32),
          pltpu.VMEM((N,), jnp.int32),
          pltpu.VMEM((SCAT_CHUNK,), jnp.int32),
          pltpu.VMEM((SCAT_CHUNK,), jnp.float32),
          pltpu.SemaphoreType.DMA,
      ],
  )
  def scatter_kernel(mT_hbm, dst_hbm, out_hbm, acc_v, tmp_v, idx_v, val_v,
                     sem):
    t = _wid()
    zeros16 = jnp.zeros((16,), jnp.float32)
    ids = lax.iota(jnp.int32, 16)

    @pl.loop(0, N // 16)
    def _(i):
      acc_v[pl.ds(i * 16, 16)] = zeros16

    @pl.loop(0, E // SCAT_CHUNK)
    def _(c):
      off = c * SCAT_CHUNK
      pltpu.sync_copy(dst_hbm.at[pl.ds(off, SCAT_CHUNK)], idx_v)
      pltpu.sync_copy(mT_hbm.at[t, pl.ds(off, SCAT_CHUNK)], val_v)

      @pl.loop(0, SCAT_CHUNK // 16)
      def _(g):
        idx = idx_v[pl.ds(g * 16, 16)]
        vals = val_v[pl.ds(g * 16, 16)]
        _scatter_group(acc_v, tmp_v, idx, vals, ids)

    pltpu.sync_copy(acc_v, out_hbm.at[t])

  return scatter_kernel(mT, dst)


def _sc_scatter_max_pool(mT, dst, starts):
  @functools.partial(
      pl.kernel,
      out_type=jax.ShapeDtypeStruct((H, B), jnp.float32),
      mesh=_vmesh(),
      compiler_params=_SC_PARAMS,
      scratch_types=[
          pltpu.VMEM((N + 16,), jnp.float32),
          pltpu.VMEM((N,), jnp.int32),
          pltpu.VMEM((SCAT_CHUNK,), jnp.int32),
          pltpu.VMEM((SCAT_CHUNK,), jnp.float32),
          pltpu.VMEM((B,), jnp.float32),
          pltpu.VMEM((80,), jnp.int32),
          pltpu.SemaphoreType.DMA,
      ],
  )
  def scatter_kernel(mT_hbm, dst_hbm, starts_hbm, out_hbm, acc_v, tmp_v,
                     idx_v, val_v, g_v, starts_v, sem):
    t = _wid()
    zeros16 = jnp.zeros((16,), jnp.float32)
    ids = lax.iota(jnp.int32, 16)
    pltpu.sync_copy(starts_hbm, starts_v)

    @pl.loop(0, (N + 16) // 16)
    def _(i):
      acc_v[pl.ds(i * 16, 16)] = zeros16

    @pl.loop(0, E // SCAT_CHUNK)
    def _(c):
      off = c * SCAT_CHUNK
      pltpu.sync_copy(dst_hbm.at[pl.ds(off, SCAT_CHUNK)], idx_v)
      pltpu.sync_copy(mT_hbm.at[t, pl.ds(off, SCAT_CHUNK)], val_v)

      @pl.loop(0, SCAT_CHUNK // 16)
      def _(g):
        idx = idx_v[pl.ds(g * 16, 16)]
        vals = val_v[pl.ds(g * 16, 16)]
        _scatter_group(acc_v, tmp_v, idx, vals, ids)

    lanes = lax.iota(jnp.int32, 16)

    for grp in range(B // 16):
      gv = zeros16
      for b2 in range(16):
        b = grp * 16 + b2
        s = starts_v[pl.ds((b // 16) * 16, 16)][b % 16]
        e = starts_v[pl.ds(((b + 1) // 16) * 16, 16)][(b + 1) % 16]
        n = e - s
        nfull = n // 16

        def seg_body(j, m, s=s):
          return jnp.maximum(m, acc_v[pl.ds(s + j * 16, 16)])

        m = lax.fori_loop(0, nfull, seg_body, zeros16)
        rem = n - nfull * 16
        v = acc_v[pl.ds(s + nfull * 16, 16)]
        m = jnp.maximum(m, jnp.where(lanes < rem, v, 0.0))
        gv = jnp.where(lanes == b2, lax.reduce_max(m, (0,)), gv)
      g_v[pl.ds(grp * 16, 16)] = gv

    pltpu.sync_copy(g_v, out_hbm.at[t])

  return scatter_kernel(mT, dst, starts)


def _tc_stats(bufs, weights):
  nb = len(bufs)

  def kernel(*refs):
    in_refs = refs[:nb]
    w_refs = refs[nb:2 * nb]
    b_ref = refs[2 * nb]
    out_ref = refs[2 * nb + 1]
    i = pl.program_id(0)
    x = b_ref[0]
    for r, w in zip(in_refs, w_refs):
      x = x + jnp.dot(r[...], w[...], preferred_element_type=jnp.float32)

    @pl.when(i == 0)
    def _():
      out_ref[...] = jnp.zeros_like(out_ref)

    s0 = jnp.sum(x, axis=0)[None]
    s1 = jnp.sum(x * x, axis=0)[None]
    out_ref[...] += jnp.concatenate(
        [s0, s1, jnp.zeros((6, H), jnp.float32)], axis=0)

  in_specs = [pl.BlockSpec((EB, b.shape[1]), lambda i: (i, 0)) for b in bufs]
  in_specs += [pl.BlockSpec(w.shape, lambda i: (0, 0)) for w in weights]
  return pl.pallas_call(
      kernel,
      grid=(E // EB,),
      in_specs=in_specs,
      out_specs=pl.BlockSpec((8, H), lambda i: (0, 0)),
      out_shape=jax.ShapeDtypeStruct((8, H), jnp.float32),
  )(*bufs, *weights)


def _tc_messages(bufs, weights, sums, gamma, beta, w2, b2):
  nb = len(bufs)

  def kernel(*refs):
    in_refs = refs[:nb]
    w_refs = refs[nb:2 * nb]
    b_ref = refs[2 * nb]
    sums_ref = refs[2 * nb + 1]
    g_ref = refs[2 * nb + 2]
    bt_ref = refs[2 * nb + 3]
    w2_ref = refs[2 * nb + 4]
    b2_ref = refs[2 * nb + 5]
    out_ref = refs[2 * nb + 6]
    x = b_ref[0]
    for r, w in zip(in_refs, w_refs):
      x = x + jnp.dot(r[...], w[...], preferred_element_type=jnp.float32)
    mu = sums_ref[0] * (1.0 / E)
    var = sums_ref[1] * (1.0 / E) - mu * mu
    scale = g_ref[0] * lax.rsqrt(var + 1e-5)
    shift = bt_ref[0] - mu * scale
    xh = jnp.maximum(x * scale + shift, 0.0)
    y = jnp.dot(xh, w2_ref[...], preferred_element_type=jnp.float32)
    y = y + b2_ref[0]
    out_ref[...] = y.T

  in_specs = [pl.BlockSpec((EB, b.shape[1]), lambda i: (i, 0)) for b in bufs]
  in_specs += [pl.BlockSpec(w.shape, lambda i: (0, 0)) for w in weights]
  in_specs += [
      pl.BlockSpec((8, H), lambda i: (0, 0)),
      pl.BlockSpec((1, H), lambda i: (0, 0)),
      pl.BlockSpec((1, H), lambda i: (0, 0)),
      pl.BlockSpec((H, H), lambda i: (0, 0)),
      pl.BlockSpec((1, H), lambda i: (0, 0)),
  ]
  return pl.pallas_call(
      kernel,
      grid=(E // EB,),
      in_specs=in_specs,
      out_specs=pl.BlockSpec((H, EB), lambda i: (0, i)),
      out_shape=jax.ShapeDtypeStruct((H, E), jnp.float32),
  )(*bufs, *weights, sums, gamma, beta, w2, b2)


def _tc_transpose(hT):
  def kernel(in_ref, out_ref):
    out_ref[...] = in_ref[...].T

  return pl.pallas_call(
      kernel,
      in_specs=[pl.BlockSpec((H, N), lambda: (0, 0))],
      out_specs=pl.BlockSpec((N, H), lambda: (0, 0)),
      out_shape=jax.ShapeDtypeStruct((N, H), jnp.float32),
  )(hT)


def _tc_classifier(gT, cls_w, cls_b):
  def kernel(g_ref, w_ref, b_ref, out_ref):
    g = g_ref[...].T
    out_ref[...] = (
        jnp.dot(g, w_ref[...], preferred_element_type=jnp.float32) + b_ref[0])

  return pl.pallas_call(
      kernel,
      in_specs=[
          pl.BlockSpec((H, B), lambda: (0, 0)),
          pl.BlockSpec((H, NC), lambda: (0, 0)),
          pl.BlockSpec((1, NC), lambda: (0, 0)),
      ],
      out_specs=pl.BlockSpec((B, NC), lambda: (0, 0)),
      out_shape=jax.ShapeDtypeStruct((B, NC), jnp.float32),
  )(gT, cls_w, cls_b)


def kernel(pos, edge_index, batch, c1_w1, c1_b1, c1_gamma, c1_beta, c1_w2,
           c1_b2, c2_w1, c2_b1, c2_gamma, c2_beta, c2_w2, c2_b2, cls_w, cls_b):
  src = edge_index[0]
  dst = edge_index[1]

  z5 = jnp.zeros((5, H), jnp.float32)
  w1a = jnp.concatenate([c1_w1[0:3] + c1_w1[3:6], z5], axis=0)
  w1b = jnp.concatenate([-c1_w1[3:6], z5], axis=0)
  w2b = jnp.concatenate([c2_w1[H:H + 3], z5], axis=0)
  w2c = -w2b
  w2a = c2_w1[0:H]
  p8 = jnp.pad(pos, ((0, 0), (0, 5)))
  starts = jnp.searchsorted(batch, jnp.arange(B + 1, dtype=jnp.int32),
                            side="left").astype(jnp.int32)
  starts = jnp.pad(starts, (0, 15), constant_values=N)

  ps8, pd8 = _sc_gather_pos(p8, src, dst)

  bufs1 = [ps8, pd8]
  ws1 = [w1a, w1b]
  sums1 = _tc_stats(bufs1, ws1 + [c1_b1[None]])
  m1T = _tc_messages(bufs1, ws1 + [c1_b1[None]], sums1, c1_gamma[None],
                     c1_beta[None], c1_w2, c1_b2[None])
  h1T = _sc_scatter_max(m1T, dst)
  h1 = _tc_transpose(h1T)

  hs = _sc_gather_rows(h1, src)
  bufs2 = [hs, ps8, pd8]
  ws2 = [w2a, w2b, w2c]
  sums2 = _tc_stats(bufs2, ws2 + [c2_b1[None]])
  m2T = _tc_messages(bufs2, ws2 + [c2_b1[None]], sums2, c2_gamma[None],
                     c2_beta[None], c2_w2, c2_b2[None])
  gT = _sc_scatter_max_pool(m2T, dst, starts)

  return _tc_classifier(gT, cls_w, cls_b[None])


# speedup vs baseline: 1.2259x; 1.2259x over previous
import dataclasses
import functools

import jax
import jax.numpy as jnp
from jax import lax
from jax.experimental import pallas as pl
from jax.experimental.pallas import tpu as pltpu
from jax.experimental.pallas import tpu_sc as plsc

N = 50000
E = 800000
H = 32
NC = 10
B = 64

NUM_CORES = 2
NUM_SUBCORES = 16
NW = NUM_CORES * NUM_SUBCORES
E_PER_W = E // NW
GATHER_CHUNK = 1000
POS_CHUNK = 5000
SCAT_CHUNK = 4000
EB = 6400

_SC_PARAMS = pltpu.CompilerParams(use_tc_tiling_on_sc=False)
if "needs_layout_passes" in pltpu.CompilerParams.__dataclass_fields__:
  _SC_PARAMS = dataclasses.replace(_SC_PARAMS, needs_layout_passes=False)


def _vmesh():
  return plsc.VectorSubcoreMesh(core_axis_name="c", subcore_axis_name="s")


def _wid():
  return lax.axis_index("s") * NUM_CORES + lax.axis_index("c")


def _sc_gather_rows(table, idx):
  D = table.shape[1]

  @functools.partial(
      pl.kernel,
      out_type=jax.ShapeDtypeStruct((E, D), jnp.float32),
      mesh=_vmesh(),
      compiler_params=_SC_PARAMS,
      scratch_types=[
          pltpu.VMEM((GATHER_CHUNK,), jnp.int32),
          pltpu.VMEM((GATHER_CHUNK, D), jnp.float32),
          pltpu.SemaphoreType.DMA,
      ],
  )
  def gather_kernel(table_hbm, idx_hbm, out_hbm, idx_v, rows_v, sem):
    base = _wid() * E_PER_W

    @pl.loop(0, E_PER_W // GATHER_CHUNK)
    def _(j):
      off = base + j * GATHER_CHUNK
      pltpu.sync_copy(idx_hbm.at[pl.ds(off, GATHER_CHUNK)], idx_v)
      pltpu.async_copy(table_hbm.at[idx_v], rows_v, sem).wait()
      pltpu.sync_copy(rows_v, out_hbm.at[pl.ds(off, GATHER_CHUNK)])

  return gather_kernel(table, idx)


def _sc_gather_pos(p8, src, dst):
  @functools.partial(
      pl.kernel,
      out_type=(
          jax.ShapeDtypeStruct((E, 8), jnp.float32),
          jax.ShapeDtypeStruct((E, 8), jnp.float32),
      ),
      mesh=_vmesh(),
      compiler_params=_SC_PARAMS,
      scratch_types=[
          pltpu.VMEM((POS_CHUNK,), jnp.int32),
          pltpu.VMEM((POS_CHUNK,), jnp.int32),
          pltpu.VMEM((POS_CHUNK, 8), jnp.float32),
          pltpu.VMEM((POS_CHUNK, 8), jnp.float32),
          pltpu.SemaphoreType.DMA,
          pltpu.SemaphoreType.DMA,
      ],
  )
  def gather_kernel(p8_hbm, src_hbm, dst_hbm, ps_hbm, pd_hbm, si_v, di_v,
                    ps_v, pd_v, sem1, sem2):
    base = _wid() * E_PER_W

    @pl.loop(0, E_PER_W // POS_CHUNK)
    def _(j):
      off = base + j * POS_CHUNK
      pltpu.sync_copy(src_hbm.at[pl.ds(off, POS_CHUNK)], si_v)
      pltpu.sync_copy(dst_hbm.at[pl.ds(off, POS_CHUNK)], di_v)
      pltpu.async_copy(p8_hbm.at[si_v], ps_v, sem1).wait()
      pltpu.async_copy(p8_hbm.at[di_v], pd_v, sem2).wait()
      pltpu.sync_copy(ps_v, ps_hbm.at[pl.ds(off, POS_CHUNK)])
      pltpu.sync_copy(pd_v, pd_hbm.at[pl.ds(off, POS_CHUNK)])

  return gather_kernel(p8, src, dst)


def _pair_rmw(acc0_v, acc1_v, idx_v, val_v, g):
  idx0 = idx_v[pl.ds(g * 32, 16)]
  vals0 = val_v[pl.ds(g * 32, 16)]
  idx1 = idx_v[pl.ds(g * 32 + 16, 16)]
  vals1 = val_v[pl.ds(g * 32 + 16, 16)]
  c0 = plsc.load_gather(acc0_v, [idx0])
  c1 = plsc.load_gather(acc1_v, [idx1])
  plsc.store_scatter(acc0_v, [idx0], jnp.maximum(c0, vals0))
  plsc.store_scatter(acc1_v, [idx1], jnp.maximum(c1, vals1))
  g0 = plsc.load_gather(acc0_v, [idx0])
  g1 = plsc.load_gather(acc1_v, [idx1])
  p0 = jnp.where(g0 < vals0, 1, 0)
  p1 = jnp.where(g1 < vals1, 1, 0)

  def cond(st):
    return (lax.reduce_max(st[0], (0,)) | lax.reduce_max(st[1], (0,))) > 0

  def body(st):
    m0 = st[0] > 0
    r0 = plsc.load_gather(acc0_v, [idx0])
    plsc.store_scatter(acc0_v, [idx0], jnp.maximum(r0, vals0), mask=m0)
    m1 = st[1] > 0
    r1 = plsc.load_gather(acc1_v, [idx1])
    plsc.store_scatter(acc1_v, [idx1], jnp.maximum(r1, vals1), mask=m1)
    q0 = plsc.load_gather(acc0_v, [idx0])
    q1 = plsc.load_gather(acc1_v, [idx1])
    return (jnp.where(m0 & (q0 < vals0), 1, 0),
            jnp.where(m1 & (q1 < vals1), 1, 0))

  lax.while_loop(cond, body, (p0, p1))


def _scatter_stream(mT_hbm, dst_hbm, t, bufs):
  nch = E // SCAT_CHUNK

  for b in range(2):
    acc, idx_v, val_v, sem_i, sem_v = bufs[b]
    off = b * SCAT_CHUNK
    pltpu.async_copy(dst_hbm.at[pl.ds(off, SCAT_CHUNK)], idx_v, sem_i)
    pltpu.async_copy(mT_hbm.at[t, pl.ds(off, SCAT_CHUNK)], val_v, sem_v)

  @pl.loop(0, nch, step=2)
  def _(c0):
    for b in range(2):
      acc, idx_v, val_v, sem_i, sem_v = bufs[b]
      c = c0 + b
      off = c * SCAT_CHUNK
      pltpu.make_async_copy(dst_hbm.at[pl.ds(off, SCAT_CHUNK)], idx_v,
                            sem_i).wait()
      pltpu.make_async_copy(mT_hbm.at[t, pl.ds(off, SCAT_CHUNK)], val_v,
                            sem_v).wait()

      @pl.loop(0, SCAT_CHUNK // 32)
      def _(g):
        _pair_rmw(bufs[b][0], bufs[1 - b][0], idx_v, val_v, g)

      @pl.when(c + 2 < nch)
      def _():
        off2 = off + 2 * SCAT_CHUNK
        pltpu.async_copy(dst_hbm.at[pl.ds(off2, SCAT_CHUNK)], idx_v, sem_i)
        pltpu.async_copy(mT_hbm.at[t, pl.ds(off2, SCAT_CHUNK)], val_v, sem_v)


def _sc_scatter_max(mT, dst):
  @functools.partial(
      pl.kernel,
      out_type=jax.ShapeDtypeStruct((H, N), jnp.float32),
      mesh=_vmesh(),
      compiler_params=_SC_PARAMS,
      scratch_types=[
          pltpu.VMEM((N,), jnp.float32),
          pltpu.VMEM((N,), jnp.float32),
          pltpu.VMEM((SCAT_CHUNK,), jnp.int32),
          pltpu.VMEM((SCAT_CHUNK,), jnp.int32),
          pltpu.VMEM((SCAT_CHUNK,), jnp.float32),
          pltpu.VMEM((SCAT_CHUNK,), jnp.float32),
          pltpu.SemaphoreType.DMA,
          pltpu.SemaphoreType.DMA,
          pltpu.SemaphoreType.DMA,
          pltpu.SemaphoreType.DMA,
      ],
  )
  def scatter_kernel(mT_hbm, dst_hbm, out_hbm, acc0_v, acc1_v, idx0_v, idx1_v,
                     val0_v, val1_v, si0, si1, sv0, sv1):
    t = _wid()
    zeros16 = jnp.zeros((16,), jnp.float32)

    @pl.loop(0, N // 16)
    def _(i):
      acc0_v[pl.ds(i * 16, 16)] = zeros16
      acc1_v[pl.ds(i * 16, 16)] = zeros16

    bufs = ((acc0_v, idx0_v, val0_v, si0, sv0),
            (acc1_v, idx1_v, val1_v, si1, sv1))
    _scatter_stream(mT_hbm, dst_hbm, t, bufs)

    @pl.loop(0, N // 16)
    def _(i):
      s = pl.ds(i * 16, 16)
      acc0_v[s] = jnp.maximum(acc0_v[s], acc1_v[s])

    pltpu.sync_copy(acc0_v, out_hbm.at[t])

  return scatter_kernel(mT, dst)


def _sc_scatter_max_pool(mT, dst, starts):
  @functools.partial(
      pl.kernel,
      out_type=jax.ShapeDtypeStruct((H, B), jnp.float32),
      mesh=_vmesh(),
      compiler_params=_SC_PARAMS,
      scratch_types=[
          pltpu.VMEM((N + 16,), jnp.float32),
          pltpu.VMEM((N + 16,), jnp.float32),
          pltpu.VMEM((SCAT_CHUNK,), jnp.int32),
          pltpu.VMEM((SCAT_CHUNK,), jnp.int32),
          pltpu.VMEM((SCAT_CHUNK,), jnp.float32),
          pltpu.VMEM((SCAT_CHUNK,), jnp.float32),
          pltpu.VMEM((B,), jnp.float32),
          pltpu.VMEM((80,), jnp.int32),
          pltpu.SemaphoreType.DMA,
          pltpu.SemaphoreType.DMA,
          pltpu.SemaphoreType.DMA,
          pltpu.SemaphoreType.DMA,
      ],
  )
  def scatter_kernel(mT_hbm, dst_hbm, starts_hbm, out_hbm, acc_v, acc1_v,
                     idx0_v, idx1_v, val0_v, val1_v, g_v, starts_v, si0, si1,
                     sv0, sv1):
    t = _wid()
    zeros16 = jnp.zeros((16,), jnp.float32)
    pltpu.sync_copy(starts_hbm, starts_v)

    @pl.loop(0, (N + 16) // 16)
    def _(i):
      acc_v[pl.ds(i * 16, 16)] = zeros16
      acc1_v[pl.ds(i * 16, 16)] = zeros16

    bufs = ((acc_v, idx0_v, val0_v, si0, sv0),
            (acc1_v, idx1_v, val1_v, si1, sv1))
    _scatter_stream(mT_hbm, dst_hbm, t, bufs)

    @pl.loop(0, N // 16)
    def _(i):
      s = pl.ds(i * 16, 16)
      acc_v[s] = jnp.maximum(acc_v[s], acc1_v[s])

    lanes = lax.iota(jnp.int32, 16)

    for grp in range(B // 16):
      gv = zeros16
      for b2 in range(16):
        b = grp * 16 + b2
        s = starts_v[pl.ds((b // 16) * 16, 16)][b % 16]
        e = starts_v[pl.ds(((b + 1) // 16) * 16, 16)][(b + 1) % 16]
        n = e - s
        nfull = n // 16

        def seg_body(j, m, s=s):
          return jnp.maximum(m, acc_v[pl.ds(s + j * 16, 16)])

        m = lax.fori_loop(0, nfull, seg_body, zeros16)
        rem = n - nfull * 16
        v = acc_v[pl.ds(s + nfull * 16, 16)]
        m = jnp.maximum(m, jnp.where(lanes < rem, v, 0.0))
        gv = jnp.where(lanes == b2, lax.reduce_max(m, (0,)), gv)
      g_v[pl.ds(grp * 16, 16)] = gv

    pltpu.sync_copy(g_v, out_hbm.at[t])

  return scatter_kernel(mT, dst, starts)


def _tc_stats(bufs, weights):
  nb = len(bufs)

  def kernel(*refs):
    in_refs = refs[:nb]
    w_refs = refs[nb:2 * nb]
    b_ref = refs[2 * nb]
    out_ref = refs[2 * nb + 1]
    i = pl.program_id(0)
    x = b_ref[0]
    for r, w in zip(in_refs, w_refs):
      x = x + jnp.dot(r[...], w[...], preferred_element_type=jnp.float32)

    @pl.when(i == 0)
    def _():
      out_ref[...] = jnp.zeros_like(out_ref)

    s0 = jnp.sum(x, axis=0)[None]
    s1 = jnp.sum(x * x, axis=0)[None]
    out_ref[...] += jnp.concatenate(
        [s0, s1, jnp.zeros((6, H), jnp.float32)], axis=0)

  in_specs = [pl.BlockSpec((EB, b.shape[1]), lambda i: (i, 0)) for b in bufs]
  in_specs += [pl.BlockSpec(w.shape, lambda i: (0, 0)) for w in weights]
  return pl.pallas_call(
      kernel,
      grid=(E // EB,),
      in_specs=in_specs,
      out_specs=pl.BlockSpec((8, H), lambda i: (0, 0)),
      out_shape=jax.ShapeDtypeStruct((8, H), jnp.float32),
  )(*bufs, *weights)


def _tc_messages(bufs, weights, sums, gamma, beta, w2, b2):
  nb = len(bufs)

  def kernel(*refs):
    in_refs = refs[:nb]
    w_refs = refs[nb:2 * nb]
    b_ref = refs[2 * nb]
    sums_ref = refs[2 * nb + 1]
    g_ref = refs[2 * nb + 2]
    bt_ref = refs[2 * nb + 3]
    w2_ref = refs[2 * nb + 4]
    b2_ref = refs[2 * nb + 5]
    out_ref = refs[2 * nb + 6]
    x = b_ref[0]
    for r, w in zip(in_refs, w_refs):
      x = x + jnp.dot(r[...], w[...], preferred_element_type=jnp.float32)
    mu = sums_ref[0] * (1.0 / E)
    var = sums_ref[1] * (1.0 / E) - mu * mu
    scale = g_ref[0] * lax.rsqrt(var + 1e-5)
    shift = bt_ref[0] - mu * scale
    xh = jnp.maximum(x * scale + shift, 0.0)
    y = jnp.dot(xh, w2_ref[...], preferred_element_type=jnp.float32)
    y = y + b2_ref[0]
    out_ref[...] = y.T

  in_specs = [pl.BlockSpec((EB, b.shape[1]), lambda i: (i, 0)) for b in bufs]
  in_specs += [pl.BlockSpec(w.shape, lambda i: (0, 0)) for w in weights]
  in_specs += [
      pl.BlockSpec((8, H), lambda i: (0, 0)),
      pl.BlockSpec((1, H), lambda i: (0, 0)),
      pl.BlockSpec((1, H), lambda i: (0, 0)),
      pl.BlockSpec((H, H), lambda i: (0, 0)),
      pl.BlockSpec((1, H), lambda i: (0, 0)),
  ]
  return pl.pallas_call(
      kernel,
      grid=(E // EB,),
      in_specs=in_specs,
      out_specs=pl.BlockSpec((H, EB), lambda i: (0, i)),
      out_shape=jax.ShapeDtypeStruct((H, E), jnp.float32),
  )(*bufs, *weights, sums, gamma, beta, w2, b2)


def _tc_transpose(hT):
  def kernel(in_ref, out_ref):
    out_ref[...] = in_ref[...].T

  return pl.pallas_call(
      kernel,
      in_specs=[pl.BlockSpec((H, N), lambda: (0, 0))],
      out_specs=pl.BlockSpec((N, H), lambda: (0, 0)),
      out_shape=jax.ShapeDtypeStruct((N, H), jnp.float32),
  )(hT)


def _tc_classifier(gT, cls_w, cls_b):
  def kernel(g_ref, w_ref, b_ref, out_ref):
    g = g_ref[...].T
    out_ref[...] = (
        jnp.dot(g, w_ref[...], preferred_element_type=jnp.float32) + b_ref[0])

  return pl.pallas_call(
      kernel,
      in_specs=[
          pl.BlockSpec((H, B), lambda: (0, 0)),
          pl.BlockSpec((H, NC), lambda: (0, 0)),
          pl.BlockSpec((1, NC), lambda: (0, 0)),
      ],
      out_specs=pl.BlockSpec((B, NC), lambda: (0, 0)),
      out_shape=jax.ShapeDtypeStruct((B, NC), jnp.float32),
  )(gT, cls_w, cls_b)


def kernel(pos, edge_index, batch, c1_w1, c1_b1, c1_gamma, c1_beta, c1_w2,
           c1_b2, c2_w1, c2_b1, c2_gamma, c2_beta, c2_w2, c2_b2, cls_w, cls_b):
  src = edge_index[0]
  dst = edge_index[1]

  z5 = jnp.zeros((5, H), jnp.float32)
  w1a = jnp.concatenate([c1_w1[0:3] + c1_w1[3:6], z5], axis=0)
  w1b = jnp.concatenate([-c1_w1[3:6], z5], axis=0)
  w2b = jnp.concatenate([c2_w1[H:H + 3], z5], axis=0)
  w2c = -w2b
  w2a = c2_w1[0:H]
  p8 = jnp.pad(pos, ((0, 0), (0, 5)))
  starts = jnp.searchsorted(batch, jnp.arange(B + 1, dtype=jnp.int32),
                            side="left").astype(jnp.int32)
  starts = jnp.pad(starts, (0, 15), constant_values=N)

  ps8, pd8 = _sc_gather_pos(p8, src, dst)

  bufs1 = [ps8, pd8]
  ws1 = [w1a, w1b]
  sums1 = _tc_stats(bufs1, ws1 + [c1_b1[None]])
  m1T = _tc_messages(bufs1, ws1 + [c1_b1[None]], sums1, c1_gamma[None],
                     c1_beta[None], c1_w2, c1_b2[None])
  h1T = _sc_scatter_max(m1T, dst)
  h1 = _tc_transpose(h1T)

  hs = _sc_gather_rows(h1, src)
  bufs2 = [hs, ps8, pd8]
  ws2 = [w2a, w2b, w2c]
  sums2 = _tc_stats(bufs2, ws2 + [c2_b1[None]])
  m2T = _tc_messages(bufs2, ws2 + [c2_b1[None]], sums2, c2_gamma[None],
                     c2_beta[None], c2_w2, c2_b2[None])
  gT = _sc_scatter_max_pool(m2T, dst, starts)

  return _tc_classifier(gT, cls_w, cls_b[None])
